# baseline (speedup 1.0000x reference)
import functools

import jax
import jax.numpy as jnp
from jax import lax
from jax.experimental import pallas as pl
from jax.experimental.pallas import tpu as pltpu
from jax.experimental.pallas import tpu_sc as plsc


@functools.cache
def _gather_row():
    mesh = plsc.VectorSubcoreMesh(core_axis_name="c", subcore_axis_name="s")

    @functools.partial(
        pl.kernel,
        mesh=mesh,
        out_type=jax.ShapeDtypeStruct((1, 128), jnp.float32),
        scratch_types=[
            pltpu.VMEM((1,), jnp.int32),
            pltpu.VMEM((1, 128), jnp.float32),
            pltpu.SemaphoreType.DMA,
        ],
    )
    def k(table_hbm, idx_hbm, out_hbm, idx_v, row_v, sem):
        wid = lax.axis_index("s") * 2 + lax.axis_index("c")

        @pl.when(wid == 0)
        def _():
            pltpu.sync_copy(idx_hbm, idx_v)
            pltpu.async_copy(table_hbm.at[idx_v], row_v, sem).wait()
            pltpu.sync_copy(row_v, out_hbm)

    return k


def kernel(x1, x2):
    return _gather_row()(x1, x2.reshape((1,)))


# speedup vs baseline: 1.0738x; 1.0738x over previous
import functools

import jax
import jax.numpy as jnp
from jax import lax
from jax.experimental import pallas as pl
from jax.experimental.pallas import tpu as pltpu
from jax.experimental.pallas import tpu_sc as plsc


@functools.cache
def _gather_row():
    mesh = plsc.VectorSubcoreMesh(
        core_axis_name="c", subcore_axis_name="s", num_cores=1, num_subcores=1
    )

    @functools.partial(
        pl.kernel,
        mesh=mesh,
        out_type=jax.ShapeDtypeStruct((1, 128), jnp.float32),
        scratch_types=[
            pltpu.VMEM((1,), jnp.int32),
            pltpu.VMEM((1, 128), jnp.float32),
            pltpu.SemaphoreType.DMA,
        ],
    )
    def k(table_hbm, idx_hbm, out_hbm, idx_v, row_v, sem):
        wid = lax.axis_index("s") * 2 + lax.axis_index("c")

        @pl.when(wid == 0)
        def _():
            pltpu.sync_copy(idx_hbm, idx_v)
            pltpu.async_copy(table_hbm.at[idx_v], row_v, sem).wait()
            pltpu.sync_copy(row_v, out_hbm)

    return k


def kernel(x1, x2):
    return _gather_row()(x1, x2.reshape((1,)))


# speedup vs baseline: 1.1634x; 1.0834x over previous
import functools

import jax
import jax.numpy as jnp
from jax import lax
from jax.experimental import pallas as pl
from jax.experimental.pallas import tpu as pltpu
from jax.experimental.pallas import tpu_sc as plsc


@functools.cache
def _gather_row():
    mesh = plsc.ScalarSubcoreMesh(axis_name="c", num_cores=1)

    @functools.partial(
        pl.kernel,
        mesh=mesh,
        out_type=jax.ShapeDtypeStruct((1, 128), jnp.float32),
        scratch_types=[
            pltpu.SMEM((1,), jnp.int32),
        ],
    )
    def k(table_hbm, idx_hbm, out_hbm, idx_s):
        pltpu.sync_copy(idx_hbm, idx_s)
        off = idx_s[0]
        pltpu.sync_copy(table_hbm.at[pl.ds(off, 1)], out_hbm)

    return k


def kernel(x1, x2):
    return _gather_row()(x1, x2.reshape((1,)))


# speedup vs baseline: 1.1808x; 1.0150x over previous
import functools

import jax
import jax.numpy as jnp
from jax.experimental import pallas as pl
from jax.experimental.pallas import tpu as pltpu
from jax.experimental.pallas import tpu_sc as plsc


@functools.cache
def _gather_row():
    mesh = plsc.ScalarSubcoreMesh(axis_name="c", num_cores=1)

    @functools.partial(
        pl.kernel,
        mesh=mesh,
        out_type=jax.ShapeDtypeStruct((1, 128), jnp.float32),
        scratch_types=[
            pltpu.SMEM((1, 1), jnp.int32),
        ],
    )
    def k(table_hbm, idx_hbm, out_hbm, idx_s):
        pltpu.sync_copy(idx_hbm, idx_s)
        off = idx_s[0, 0]
        pltpu.sync_copy(table_hbm.at[pl.ds(off, 1)], out_hbm)

    return k


def kernel(x1, x2):
    return _gather_row()(x1, x2)


# speedup vs baseline: 10.9556x; 9.2779x over previous
import jax
import jax.numpy as jnp
from jax.experimental import pallas as pl
from jax.experimental.pallas import tpu as pltpu


def _copy_row(idx_ref, x_ref, o_ref):
    r = idx_ref[0] % 8
    o_ref[...] = x_ref[pl.ds(r, 1), :]


def kernel(x1, x2):
    grid_spec = pltpu.PrefetchScalarGridSpec(
        num_scalar_prefetch=1,
        grid=(1,),
        in_specs=[pl.BlockSpec((8, 128), lambda i, idx: (idx[0] // 8, 0))],
        out_specs=pl.BlockSpec((1, 128), lambda i, idx: (0, 0)),
    )
    return pl.pallas_call(
        _copy_row,
        grid_spec=grid_spec,
        out_shape=jax.ShapeDtypeStruct((1, 128), jnp.float32),
    )(x2.reshape((1,)), x1)
